# Initial kernel scaffold; baseline (speedup 1.0000x reference)
import jax
import jax.numpy as jnp
from jax.experimental import pallas as pl


def kernel(vertices, img_feats, proj_mat):
    raise NotImplementedError("write your pallas kernel here")


# speedup vs baseline: 1.3927x; 1.3927x over previous
import functools

import jax
import jax.numpy as jnp
from jax import lax
from jax.experimental import pallas as pl
from jax.experimental.pallas import tpu as pltpu
from jax.experimental.pallas import tpu_sc as plsc

L = 16
NC = 2
NS = 16
NW = NC * NS

B = 8
N = 8192
C = 384
H = 96
W = 96
PW = (B * N) // NW
NG = PW // L
ROWS = B * H * W
WPB = N // PW
CG = C // L


def _body(verts, img_rows, proj, out, stage, verts_v, xyz_v, idx_v, w_v,
          gbuf0, gbuf1, obuf0, obuf1, pvec, mm_v, grp_v,
          sem0, sem1):
    c = lax.axis_index("c")
    s = lax.axis_index("s")
    wid = c * NS + s
    b = wid // WPB
    noff = (wid % WPB) * PW
    row0 = wid * PW

    pltpu.sync_copy(verts.at[b, :, pl.ds(noff, PW)], verts_v)
    pltpu.sync_copy(proj.at[b], pvec)

    p00, p01, p02, p03 = pvec[0], pvec[1], pvec[2], pvec[3]
    p10, p11, p12, p13 = pvec[4], pvec[5], pvec[6], pvec[7]
    p20, p21, p22, p23 = pvec[8], pvec[9], pvec[10], pvec[11]
    p30, p31, p32, p33 = pvec[12], pvec[13], pvec[14], pvec[15]

    def pa(g, carry):
        mn, mx = carry
        sl = pl.ds(g * L, L)
        vx = verts_v[0, sl]
        vy = verts_v[1, sl]
        vz = verts_v[2, sl]
        cx = p00 * vx + p01 * vy + p02 * vz + p03
        cy = p10 * vx + p11 * vy + p12 * vz + p13
        cz = p20 * vx + p21 * vy + p22 * vz + p23
        cw = p30 * vx + p31 * vy + p32 * vz + p33
        x = (cx / cw + 1.0) * 0.5
        y = 1.0 - (cy / cw + 1.0) * 0.5
        z = cz / cw
        xyz_v[0, sl] = jnp.clip(x * float(W), 0.0, float(W - 1))
        xyz_v[1, sl] = jnp.clip(y * float(H), 0.0, float(H - 1))
        xyz_v[2, sl] = z
        return jnp.minimum(mn, z), jnp.maximum(mx, z)

    big = jnp.full((L,), 3.0e38, jnp.float32)
    mn, mx = lax.fori_loop(0, NG, pa, (big, -big))

    lanes = lax.iota(jnp.int32, L)
    zeros = jnp.zeros((L,), jnp.int32)

    def lane_reduce(v, op):
        for sh in (8, 4, 2, 1):
            mm_v[0] = v
            perm = plsc.load_gather(mm_v, [zeros, lanes ^ sh])
            v = op(v, perm)
        return v

    rmn = lane_reduce(mn, jnp.minimum)
    rmx = lane_reduce(mx, jnp.maximum)
    mm_v[0] = rmn
    mm_v[1] = rmx
    pltpu.sync_copy(mm_v, stage.at[wid])
    plsc.subcore_barrier()
    pltpu.sync_copy(stage.at[pl.ds((wid // WPB) * WPB, WPB)], grp_v)
    bmn = jnp.minimum(jnp.minimum(grp_v[0, 0], grp_v[1, 0]),
                      jnp.minimum(grp_v[2, 0], grp_v[3, 0]))
    bmx = jnp.maximum(jnp.maximum(grp_v[0, 1], grp_v[1, 1]),
                      jnp.maximum(grp_v[2, 1], grp_v[3, 1]))
    inv = 1.0 / (bmx - bmn)

    base = b * (H * W)

    def pb(g, _):
        sl = pl.ds(g * L, L)
        x0 = xyz_v[0, sl]
        y0 = xyz_v[1, sl]
        z = xyz_v[2, sl]
        x1i = x0.astype(jnp.int32)
        x1f = x1i.astype(jnp.float32)
        x2i = jnp.where(x0 == x1f, x1i, x1i + 1)
        x2f = x2i.astype(jnp.float32)
        y1i = y0.astype(jnp.int32)
        y1f = y1i.astype(jnp.float32)
        y2i = jnp.where(y0 == y1f, y1i, y1i + 1)
        y2f = y2i.astype(jnp.float32)
        zw = jnp.exp((bmn - z) * inv)
        ax = x0 - x1f
        bx = x2f - x0
        ay = y0 - y1f
        by = y2f - y0
        w_v[g, pl.ds(0, L)] = bx * by * zw
        w_v[g, pl.ds(L, L)] = ax * by * zw
        w_v[g, pl.ds(2 * L, L)] = bx * ay * zw
        w_v[g, pl.ds(3 * L, L)] = ax * ay * zw
        r1 = base + y1i * W
        r2 = base + y2i * W
        idx_v[g, pl.ds(0, L)] = jnp.clip(r1 + x1i, 0, ROWS - 1)
        idx_v[g, pl.ds(L, L)] = jnp.clip(r1 + x2i, 0, ROWS - 1)
        idx_v[g, pl.ds(2 * L, L)] = jnp.clip(r2 + x1i, 0, ROWS - 1)
        idx_v[g, pl.ds(3 * L, L)] = jnp.clip(r2 + x2i, 0, ROWS - 1)
        return 0

    lax.fori_loop(0, NG, pb, 0)

    def gather(g, gb, sem):
        pltpu.async_copy(img_rows.at[idx_v.at[g]], gb, sem)

    def gwait(g, gb, sem):
        pltpu.make_async_copy(img_rows.at[idx_v.at[g]], gb, sem).wait()

    def blend(g, gb, ob):
        def pbody(p, _):
            gi = jnp.full((L,), g, jnp.int32)
            w11 = plsc.load_gather(w_v, [gi, jnp.full((L,), p, jnp.int32)])
            w21 = plsc.load_gather(w_v, [gi, jnp.full((L,), p + L, jnp.int32)])
            w12 = plsc.load_gather(
                w_v, [gi, jnp.full((L,), p + 2 * L, jnp.int32)])
            w22 = plsc.load_gather(
                w_v, [gi, jnp.full((L,), p + 3 * L, jnp.int32)])
            for cg in range(CG):
                sl = pl.ds(cg * L, L)
                ob[p, sl] = (w11 * gb[p, sl] + w21 * gb[p + L, sl]
                             + w12 * gb[p + 2 * L, sl]
                             + w22 * gb[p + 3 * L, sl])
            return 0

        lax.fori_loop(0, L, pbody, 0)

    gather(0, gbuf0, sem0)

    def pc(i, _):
        g0 = 2 * i
        g1 = 2 * i + 1
        gwait(g0, gbuf0, sem0)
        gather(g1, gbuf1, sem1)
        blend(g0, gbuf0, obuf0)
        pltpu.sync_copy(obuf0, out.at[pl.ds(row0 + g0 * L, L)])
        gwait(g1, gbuf1, sem1)

        @pl.when(i < NG // 2 - 1)
        def _():
            gather(g0 + 2, gbuf0, sem0)

        blend(g1, gbuf1, obuf1)
        pltpu.sync_copy(obuf1, out.at[pl.ds(row0 + g1 * L, L)])
        return 0

    lax.fori_loop(0, NG // 2, pc, 0)


@jax.jit
def _run(verts_t, img_rows, proj_splat):
    mesh = plsc.VectorSubcoreMesh(core_axis_name="c", subcore_axis_name="s",
                                  num_cores=NC, num_subcores=NS)
    fn = pl.kernel(
        _body,
        out_type=(jax.ShapeDtypeStruct((B * N, C), jnp.float32),
                  jax.ShapeDtypeStruct((NW, 2, L), jnp.float32)),
        mesh=mesh,
        compiler_params=pltpu.CompilerParams(needs_layout_passes=False),
        scratch_types=[
            pltpu.VMEM((3, PW), jnp.float32),
            pltpu.VMEM((3, PW), jnp.float32),
            pltpu.VMEM((NG, 4 * L), jnp.int32),
            pltpu.VMEM((NG, 4 * L), jnp.float32),
            pltpu.VMEM((4 * L, C), jnp.float32),
            pltpu.VMEM((4 * L, C), jnp.float32),
            pltpu.VMEM((L, C), jnp.float32),
            pltpu.VMEM((L, C), jnp.float32),
            pltpu.VMEM((16, L), jnp.float32),
            pltpu.VMEM((2, L), jnp.float32),
            pltpu.VMEM((WPB, 2, L), jnp.float32),
            pltpu.SemaphoreType.DMA,
            pltpu.SemaphoreType.DMA,
        ],
    )
    return fn(verts_t, img_rows, proj_splat)


def kernel(vertices, img_feats, proj_mat):
    img_rows = img_feats.transpose(0, 2, 3, 1).reshape(B * H * W, C)
    verts_t = vertices.transpose(0, 2, 1)
    proj_splat = jnp.broadcast_to(
        proj_mat.reshape(B, 16)[:, :, None], (B, 16, L))
    out, _ = _run(verts_t, img_rows, proj_splat)
    return out.reshape(B, N, C)


# speedup vs baseline: 3.1290x; 2.2467x over previous
import functools

import jax
import jax.numpy as jnp
from jax import lax
from jax.experimental import pallas as pl
from jax.experimental.pallas import tpu as pltpu
from jax.experimental.pallas import tpu_sc as plsc

L = 16
NC = 2
NS = 16
NW = NC * NS

B = 8
N = 8192
C = 384
H = 96
W = 96
PW = (B * N) // NW
NG = PW // L
ROWS = B * H * W
WPB = N // PW
CG = C // L


def _body(verts, img_rows, proj, out, stage, verts_v, xyz_v, idx_v, w_v,
          pos_v, zbuf, gbuf0, gbuf1, obuf0, obuf1, pvec, mm_v, grp_v,
          sem0, sem1, osem0, osem1):
    c = lax.axis_index("c")
    s = lax.axis_index("s")
    wid = c * NS + s
    b = wid // WPB
    noff = (wid % WPB) * PW
    row0 = wid * PW

    pltpu.sync_copy(verts.at[b, :, pl.ds(noff, PW)], verts_v)
    pltpu.sync_copy(proj.at[b], pvec)

    def bf16r(v):
        u = plsc.bitcast(v, jnp.int32)
        bias = 0x7FFF + ((u >> 16) & 1)
        return plsc.bitcast((u + bias) & jnp.int32(-65536), jnp.float32)

    p00, p01, p02, p03 = (bf16r(pvec[k]) for k in range(0, 4))
    p10, p11, p12, p13 = (bf16r(pvec[k]) for k in range(4, 8))
    p20, p21, p22, p23 = (bf16r(pvec[k]) for k in range(8, 12))
    p30, p31, p32, p33 = (bf16r(pvec[k]) for k in range(12, 16))

    def pa(g, carry):
        mn, mx = carry
        sl = pl.ds(g * L, L)
        vx = bf16r(verts_v[0, sl])
        vy = bf16r(verts_v[1, sl])
        vz = bf16r(verts_v[2, sl])
        cx = p00 * vx + p01 * vy + p02 * vz + p03
        cy = p10 * vx + p11 * vy + p12 * vz + p13
        cz = p20 * vx + p21 * vy + p22 * vz + p23
        cw = p30 * vx + p31 * vy + p32 * vz + p33
        x = (cx / cw + 1.0) * 0.5
        y = 1.0 - (cy / cw + 1.0) * 0.5
        z = cz / cw
        xyz_v[0, sl] = jnp.clip(x * float(W), 0.0, float(W - 1))
        xyz_v[1, sl] = jnp.clip(y * float(H), 0.0, float(H - 1))
        xyz_v[2, sl] = z
        return jnp.minimum(mn, z), jnp.maximum(mx, z)

    big = jnp.full((L,), 3.0e38, jnp.float32)
    mn, mx = lax.fori_loop(0, NG, pa, (big, -big))

    lanes = lax.iota(jnp.int32, L)
    zeros = jnp.zeros((L,), jnp.int32)

    def lane_reduce(v, op):
        for sh in (8, 4, 2, 1):
            mm_v[0] = v
            perm = plsc.load_gather(mm_v, [zeros, lanes ^ sh])
            v = op(v, perm)
        return v

    rmn = lane_reduce(mn, jnp.minimum)
    rmx = lane_reduce(mx, jnp.maximum)
    mm_v[0] = rmn
    mm_v[1] = rmx
    pltpu.sync_copy(mm_v, stage.at[wid])
    plsc.subcore_barrier()
    pltpu.sync_copy(stage.at[pl.ds((wid // WPB) * WPB, WPB)], grp_v)
    bmn = jnp.minimum(jnp.minimum(grp_v[0, 0], grp_v[1, 0]),
                      jnp.minimum(grp_v[2, 0], grp_v[3, 0]))
    bmx = jnp.maximum(jnp.maximum(grp_v[0, 1], grp_v[1, 1]),
                      jnp.maximum(grp_v[2, 1], grp_v[3, 1]))
    rng = bmx - bmn

    base = b * (H * W)

    def pb(g, carry):
        cnt, zcnt = carry
        sl = pl.ds(g * L, L)
        x0 = xyz_v[0, sl]
        y0 = xyz_v[1, sl]
        z = xyz_v[2, sl]
        x1i = x0.astype(jnp.int32)
        x1f = x1i.astype(jnp.float32)
        x2i = jnp.where(x0 == x1f, x1i, x1i + 1)
        x2f = x2i.astype(jnp.float32)
        y1i = y0.astype(jnp.int32)
        y1f = y1i.astype(jnp.float32)
        y2i = jnp.where(y0 == y1f, y1i, y1i + 1)
        y2f = y2i.astype(jnp.float32)
        zw = jnp.exp((bmn - z) / rng)
        ax = x0 - x1f
        bx = x2f - x0
        ay = y0 - y1f
        by = y2f - y0
        w_v[g, pl.ds(0, L)] = bx * by * zw
        w_v[g, pl.ds(L, L)] = ax * by * zw
        w_v[g, pl.ds(2 * L, L)] = bx * ay * zw
        w_v[g, pl.ds(3 * L, L)] = ax * ay * zw
        r1 = base + y1i * W
        r2 = base + y2i * W
        idx_v[g, pl.ds(0, L)] = jnp.clip(r1 + x1i, 0, ROWS - 1)
        idx_v[g, pl.ds(L, L)] = jnp.clip(r1 + x2i, 0, ROWS - 1)
        idx_v[g, pl.ds(2 * L, L)] = jnp.clip(r2 + x1i, 0, ROWS - 1)
        idx_v[g, pl.ds(3 * L, L)] = jnp.clip(r2 + x2i, 0, ROWS - 1)
        pid = g * L + lanes
        mask = jnp.logical_and(x1i != x2i, y1i != y2i)
        fsum = plsc.cumsum(mask.astype(jnp.int32))
        plsc.store_scatter(pos_v, [cnt + fsum - 1], pid, mask=mask)
        nmask = jnp.logical_not(mask)
        zsum = plsc.cumsum(nmask.astype(jnp.int32))
        plsc.store_scatter(pos_v, [(PW - 1) - (zcnt + zsum - 1)], pid,
                           mask=nmask)
        return (cnt + plsc.all_reduce_population_count(mask),
                zcnt + plsc.all_reduce_population_count(nmask))

    n_vec, _ = lax.fori_loop(0, NG, pb, (zeros, zeros))

    def zb(r, _):
        for cg in range(CG):
            zbuf[r, pl.ds(cg * L, L)] = jnp.zeros((L,), jnp.float32)
        return 0

    lax.fori_loop(0, L, zb, 0)

    def blend_c(g, gb, ob):
        def pbody(p, _):
            pid = plsc.load_gather(
                pos_v, [jnp.full((L,), g * L + p, jnp.int32)])
            hi = pid >> 4
            lo = pid & (L - 1)
            w11 = plsc.load_gather(w_v, [hi, lo])
            w21 = plsc.load_gather(w_v, [hi, lo + L])
            w12 = plsc.load_gather(w_v, [hi, lo + 2 * L])
            w22 = plsc.load_gather(w_v, [hi, lo + 3 * L])
            for cg in range(CG):
                sl = pl.ds(cg * L, L)
                ob[p, sl] = (w11 * gb[p, sl] + w21 * gb[p + L, sl]
                             + w12 * gb[p + 2 * L, sl]
                             + w22 * gb[p + 3 * L, sl])
            return 0

        lax.fori_loop(0, L, pbody, 0)

    def owait(ob, osem):
        pltpu.make_async_copy(ob, out.at[pl.ds(row0, L)], osem).wait()

    def chunk(g, gb, sem, ob, osem):
        pidv = pos_v[pl.ds(g * L, L)]
        rows_out = row0 + pidv
        hi = pidv >> 4
        lo = pidv & (L - 1)
        live = jnp.any(n_vec > g * L)

        @pl.when(live)
        def _():
            for q in range(4):
                rq = plsc.load_gather(idx_v, [hi, lo + q * L])
                pltpu.async_copy(
                    img_rows.at[rq], gb.at[pl.ds(q * L, L)], sem)
            for q in range(4):
                rq = plsc.load_gather(idx_v, [hi, lo + q * L])
                pltpu.make_async_copy(
                    img_rows.at[rq], gb.at[pl.ds(q * L, L)], sem).wait()
            blend_c(g, gb, ob)
            pltpu.async_copy(ob, out.at[rows_out], osem)

        @pl.when(jnp.logical_not(live))
        def _():
            pltpu.async_copy(zbuf, out.at[rows_out], osem)

    def pc(i, _):
        @pl.when(i > 0)
        def _():
            owait(obuf0, osem0)

        chunk(2 * i, gbuf0, sem0, obuf0, osem0)

        @pl.when(i > 0)
        def _():
            owait(obuf1, osem1)

        chunk(2 * i + 1, gbuf1, sem1, obuf1, osem1)
        return 0

    lax.fori_loop(0, NG // 2, pc, 0)
    owait(obuf0, osem0)
    owait(obuf1, osem1)


@jax.jit
def _run(verts_t, img_rows, proj_splat):
    mesh = plsc.VectorSubcoreMesh(core_axis_name="c", subcore_axis_name="s",
                                  num_cores=NC, num_subcores=NS)
    fn = pl.kernel(
        _body,
        out_type=(jax.ShapeDtypeStruct((B * N, C), jnp.float32),
                  jax.ShapeDtypeStruct((NW, 2, L), jnp.float32)),
        mesh=mesh,
        compiler_params=pltpu.CompilerParams(needs_layout_passes=False),
        scratch_types=[
            pltpu.VMEM((3, PW), jnp.float32),
            pltpu.VMEM((3, PW), jnp.float32),
            pltpu.VMEM((NG, 4 * L), jnp.int32),
            pltpu.VMEM((NG, 4 * L), jnp.float32),
            pltpu.VMEM((PW,), jnp.int32),
            pltpu.VMEM((L, C), jnp.float32),
            pltpu.VMEM((4 * L, C), jnp.float32),
            pltpu.VMEM((4 * L, C), jnp.float32),
            pltpu.VMEM((L, C), jnp.float32),
            pltpu.VMEM((L, C), jnp.float32),
            pltpu.VMEM((16, L), jnp.float32),
            pltpu.VMEM((2, L), jnp.float32),
            pltpu.VMEM((WPB, 2, L), jnp.float32),
            pltpu.SemaphoreType.DMA,
            pltpu.SemaphoreType.DMA,
            pltpu.SemaphoreType.DMA,
            pltpu.SemaphoreType.DMA,
        ],
    )
    return fn(verts_t, img_rows, proj_splat)


def kernel(vertices, img_feats, proj_mat):
    img_rows = img_feats.transpose(0, 2, 3, 1).reshape(B * H * W, C)
    verts_t = vertices.transpose(0, 2, 1)
    proj_splat = jnp.broadcast_to(
        proj_mat.reshape(B, 16)[:, :, None], (B, 16, L))
    out, _ = _run(verts_t, img_rows, proj_splat)
    return out.reshape(B, N, C)
